# baseline (speedup 1.0000x reference)
import functools
import jax
import jax.numpy as jnp
from jax import lax
from jax.experimental import pallas as pl
from jax.experimental.pallas import tpu as pltpu
from jax.experimental.pallas import tpu_sc as plsc

_B, _T, _H, _NB = 16, 4096, 512, 512


def _pool_body(time_ref, feat_ref, out_ref, cnt_ref):
    tm = time_ref[0, 0, :]
    oh = (tm[:, None] == lax.broadcasted_iota(jnp.int32, (_T, _NB), 1)
          ).astype(jnp.bfloat16)
    sums = lax.dot_general(oh, feat_ref[0].astype(jnp.bfloat16),
                           (((0,), (0,)), ((), ())),
                           preferred_element_type=jnp.float32)
    ones = jnp.ones((8, _T), dtype=jnp.bfloat16)
    cnt = lax.dot_general(ones, oh, (((1,), (0,)), ((), ())),
                          preferred_element_type=jnp.float32)[0]
    cnt_ref[0, 0] = cnt
    out_ref[0] = sums / jnp.maximum(cnt, 1.0)[:, None]


def _tc_pool(time3, feat):
    return pl.pallas_call(
        _pool_body,
        grid=(_B,),
        in_specs=[
            pl.BlockSpec((1, 1, _T), lambda b: (b, 0, 0)),
            pl.BlockSpec((1, _T, _H), lambda b: (b, 0, 0)),
        ],
        out_specs=[
            pl.BlockSpec((1, _NB, _H), lambda b: (b, 0, 0)),
            pl.BlockSpec((1, 1, _NB), lambda b: (b, 0, 0)),
        ],
        out_shape=[
            jax.ShapeDtypeStruct((_B, _NB, _H), jnp.float32),
            jax.ShapeDtypeStruct((_B, 1, _NB), jnp.float32),
        ],
        compiler_params=pltpu.CompilerParams(
            dimension_semantics=("arbitrary",)),
    )(time3, feat)


def _sc_body(time_hbm, cnt_hbm, tbuf, cbuf):
    half = lax.axis_index("c")
    b = lax.axis_index("s")
    pltpu.sync_copy(time_hbm.at[b], tbuf)
    iota16 = lax.iota(jnp.int32, 16)
    n0 = half * (_NB // 2)

    def lower_bound(tgt):
        lo = jnp.zeros((16,), jnp.int32)
        hi = jnp.full((16,), _T, jnp.int32)
        for _ in range(12):
            mid = lax.shift_right_logical(lo + hi, 1)
            v = plsc.load_gather(tbuf, [mid])
            lt = v < tgt
            lo = jnp.where(lt, mid + 1, lo)
            hi = jnp.where(lt, hi, mid)
        return lo

    def do_group(j, _):
        tgt = n0 + j * 16 + iota16
        cnt = lower_bound(tgt + 1) - lower_bound(tgt)
        cbuf[pl.ds(pl.multiple_of(j * 16, 16), 16)] = cnt
        return 0
    lax.fori_loop(0, _NB // 2 // 16, do_group, 0)
    pltpu.sync_copy(cbuf, cnt_hbm.at[b, pl.ds(n0, _NB // 2)])


def _sc_counts(marked_time):
    mesh = plsc.VectorSubcoreMesh(core_axis_name="c", subcore_axis_name="s")
    f = pl.kernel(
        _sc_body,
        out_type=jax.ShapeDtypeStruct((_B, _NB), jnp.int32),
        mesh=mesh,
        scratch_types=[
            pltpu.VMEM((_T,), jnp.int32),
            pltpu.VMEM((_NB // 2,), jnp.int32),
        ],
        compiler_params=pltpu.CompilerParams(needs_layout_passes=False),
    )
    return f(marked_time)


def kernel(backbone_features, time, temporal_padding_mask):
    marked = jnp.where(temporal_padding_mask, _NB, time).astype(jnp.int32)
    cnt_sc = _sc_counts(marked)
    time3 = marked.reshape(_B, 1, _T)
    pooled, _ = _tc_pool(time3, backbone_features)
    new_padding_mask = cnt_sc == 0
    return pooled, new_padding_mask


# speedup vs baseline: 1.0004x; 1.0004x over previous
import functools
import jax
import jax.numpy as jnp
from jax import lax
from jax.experimental import pallas as pl
from jax.experimental.pallas import tpu as pltpu
from jax.experimental.pallas import tpu_sc as plsc

_B, _T, _H, _NB = 16, 4096, 512, 512


def _pool_body(time_ref, feat_ref, out_ref, cnt_ref):
    tm = time_ref[0, 0, :]
    oh = (tm[:, None] == lax.broadcasted_iota(jnp.int32, (_T, _NB), 1)
          ).astype(jnp.bfloat16)
    sums = lax.dot_general(oh, feat_ref[0].astype(jnp.bfloat16),
                           (((0,), (0,)), ((), ())),
                           preferred_element_type=jnp.float32)
    ones = jnp.ones((8, _T), dtype=jnp.bfloat16)
    cnt = lax.dot_general(ones, oh, (((1,), (0,)), ((), ())),
                          preferred_element_type=jnp.float32)[0]
    cnt_ref[0, 0] = cnt
    out_ref[0] = sums / jnp.maximum(cnt, 1.0)[:, None]


def _tc_pool(time3, feat):
    return pl.pallas_call(
        _pool_body,
        grid=(_B,),
        in_specs=[
            pl.BlockSpec((1, 1, _T), lambda b: (b, 0, 0)),
            pl.BlockSpec((1, _T, _H), lambda b: (b, 0, 0)),
        ],
        out_specs=[
            pl.BlockSpec((1, _NB, _H), lambda b: (b, 0, 0)),
            pl.BlockSpec((1, 1, _NB), lambda b: (b, 0, 0)),
        ],
        out_shape=[
            jax.ShapeDtypeStruct((_B, _NB, _H), jnp.float32),
            jax.ShapeDtypeStruct((_B, 1, _NB), jnp.float32),
        ],
        compiler_params=pltpu.CompilerParams(
            dimension_semantics=("arbitrary",)),
        cost_estimate=pl.CostEstimate(
            flops=2 * _B * _T * _NB * (_H + 8),
            bytes_accessed=(_B * _T * _H + _B * _NB * _H) * 4,
            transcendentals=0),
    )(time3, feat)


def _sc_body(time_hbm, cnt_hbm, tbuf, cbuf):
    half = lax.axis_index("c")
    b = lax.axis_index("s")
    pltpu.sync_copy(time_hbm.at[b], tbuf)
    iota16 = lax.iota(jnp.int32, 16)
    n0 = half * (_NB // 2)

    def lower_bound(tgt):
        lo = jnp.zeros((16,), jnp.int32)
        hi = jnp.full((16,), _T, jnp.int32)
        for _ in range(12):
            mid = lax.shift_right_logical(lo + hi, 1)
            v = plsc.load_gather(tbuf, [mid])
            lt = v < tgt
            lo = jnp.where(lt, mid + 1, lo)
            hi = jnp.where(lt, hi, mid)
        return lo

    def do_group(j, _):
        tgt = n0 + j * 16 + iota16
        cnt = lower_bound(tgt + 1) - lower_bound(tgt)
        cbuf[pl.ds(pl.multiple_of(j * 16, 16), 16)] = cnt
        return 0
    lax.fori_loop(0, _NB // 2 // 16, do_group, 0)
    pltpu.sync_copy(cbuf, cnt_hbm.at[b, pl.ds(n0, _NB // 2)])


def _sc_counts(marked_time):
    mesh = plsc.VectorSubcoreMesh(core_axis_name="c", subcore_axis_name="s")
    f = pl.kernel(
        _sc_body,
        out_type=jax.ShapeDtypeStruct((_B, _NB), jnp.int32),
        mesh=mesh,
        scratch_types=[
            pltpu.VMEM((_T,), jnp.int32),
            pltpu.VMEM((_NB // 2,), jnp.int32),
        ],
        compiler_params=pltpu.CompilerParams(needs_layout_passes=False),
        cost_estimate=pl.CostEstimate(
            flops=0, bytes_accessed=_B * _T * 4, transcendentals=0),
    )
    return f(marked_time)


def kernel(backbone_features, time, temporal_padding_mask):
    marked = jnp.where(temporal_padding_mask, _NB, time).astype(jnp.int32)
    cnt_sc = _sc_counts(marked)
    time3 = marked.reshape(_B, 1, _T)
    pooled, _ = _tc_pool(time3, backbone_features)
    new_padding_mask = cnt_sc == 0
    return pooled, new_padding_mask


# speedup vs baseline: 1.0967x; 1.0963x over previous
import functools
import jax
import jax.numpy as jnp
from jax import lax
from jax.experimental import pallas as pl
from jax.experimental.pallas import tpu as pltpu
from jax.experimental.pallas import tpu_sc as plsc

_B, _T, _H, _NB = 16, 4096, 512, 512


def _pool_body(time_ref, feat_ref, out_ref, cnt_ref):
    tm = time_ref[0, 0, :]
    oh = (tm[:, None] == lax.broadcasted_iota(jnp.int32, (_T, _NB), 1)
          ).astype(jnp.bfloat16)
    sums = lax.dot_general(oh, feat_ref[0].astype(jnp.bfloat16),
                           (((0,), (0,)), ((), ())),
                           preferred_element_type=jnp.float32)
    ones = jnp.ones((8, _T), dtype=jnp.bfloat16)
    cnt = lax.dot_general(ones, oh, (((1,), (0,)), ((), ())),
                          preferred_element_type=jnp.float32)[0]
    cnt_ref[0, 0] = cnt
    out_ref[0] = sums / jnp.maximum(cnt, 1.0)[:, None]


def _tc_pool(time3, feat):
    return pl.pallas_call(
        _pool_body,
        grid=(_B,),
        in_specs=[
            pl.BlockSpec((1, 1, _T), lambda b: (b, 0, 0)),
            pl.BlockSpec((1, _T, _H), lambda b: (b, 0, 0)),
        ],
        out_specs=[
            pl.BlockSpec((1, _NB, _H), lambda b: (b, 0, 0)),
            pl.BlockSpec((1, 1, _NB), lambda b: (b, 0, 0)),
        ],
        out_shape=[
            jax.ShapeDtypeStruct((_B, _NB, _H), jnp.float32),
            jax.ShapeDtypeStruct((_B, 1, _NB), jnp.float32),
        ],
        compiler_params=pltpu.CompilerParams(
            dimension_semantics=("arbitrary",)),
        cost_estimate=pl.CostEstimate(
            flops=2 * _B * _T * _NB * (_H + 8),
            bytes_accessed=(_B * _T * _H + _B * _NB * _H) * 4,
            transcendentals=0),
    )(time3, feat)


def _sc_body(time_hbm, cnt_hbm, tbuf, cbuf):
    half = lax.axis_index("c")
    b = lax.axis_index("s")
    pltpu.sync_copy(time_hbm.at[b], tbuf)
    iota16 = lax.iota(jnp.int32, 16)
    n0 = half * (_NB // 2)

    def lower_bound(tgt):
        lo = jnp.zeros((16,), jnp.int32)
        hi = jnp.full((16,), _T, jnp.int32)
        for _ in range(12):
            mid = lax.shift_right_logical(lo + hi, 1)
            v = plsc.load_gather(tbuf, [mid])
            lt = v < tgt
            lo = jnp.where(lt, mid + 1, lo)
            hi = jnp.where(lt, hi, mid)
        return lo

    def do_group(j, _):
        tgt = n0 + j * 16 + iota16
        cnt = lower_bound(tgt + 1) - lower_bound(tgt)
        cbuf[pl.ds(pl.multiple_of(j * 16, 16), 16)] = cnt
        return 0
    lax.fori_loop(0, _NB // 2 // 16, do_group, 0)
    pltpu.sync_copy(cbuf, cnt_hbm.at[b, pl.ds(n0, _NB // 2)])


def _sc_counts(marked_time):
    mesh = plsc.VectorSubcoreMesh(core_axis_name="c", subcore_axis_name="s")
    f = pl.kernel(
        _sc_body,
        out_type=jax.ShapeDtypeStruct((_B, _NB), jnp.int32),
        mesh=mesh,
        scratch_types=[
            pltpu.VMEM((_T,), jnp.int32),
            pltpu.VMEM((_NB // 2,), jnp.int32),
        ],
        compiler_params=pltpu.CompilerParams(needs_layout_passes=False),
        cost_estimate=pl.CostEstimate(
            flops=0, bytes_accessed=_B * _T * 4, transcendentals=0),
    )
    return f(marked_time)


def kernel(backbone_features, time, temporal_padding_mask):
    marked = jnp.where(temporal_padding_mask, _NB, time).astype(jnp.int32)
    time3 = marked.reshape(_B, 1, _T)
    pooled, _ = _tc_pool(time3, backbone_features)
    cnt_sc = _sc_counts(marked)
    new_padding_mask = cnt_sc == 0
    return pooled, new_padding_mask
